# baseline (speedup 1.0000x reference)
import functools

import jax
import jax.numpy as jnp
from jax import lax
from jax.experimental import pallas as pl
from jax.experimental.pallas import tpu as pltpu
from jax.experimental.pallas import tpu_sc as plsc

VOCAB = 1000000
DIM = 64
R = 16
B = 16384
L = 20
N = B * L

_info = plsc.get_sparse_core_info()
NC = _info.num_cores
NS = _info.num_subcores
NW = NC * NS
PW = N // NW
CK = 1024
NCH = PW // CK
SG = 128
NSG = CK // SG

_mesh = plsc.VectorSubcoreMesh(core_axis_name="c", subcore_axis_name="s")


@functools.partial(
    pl.kernel,
    mesh=_mesh,
    out_type=(
        jax.ShapeDtypeStruct((N, DIM), jnp.float32),
        jax.ShapeDtypeStruct((N, R), jnp.float32),
    ),
    scratch_types=[
        pltpu.VMEM((PW,), jnp.int32),
        pltpu.VMEM((CK, DIM), jnp.float32),
        pltpu.VMEM((CK, R), jnp.float32),
        pltpu.SemaphoreType.DMA,
    ],
    compiler_params=pltpu.CompilerParams(use_tc_tiling_on_sc=False),
)
def _sc_gather(table_hbm, u_hbm, idx_hbm, outb_hbm, outu_hbm,
               idx_v, rowsb_v, rowsu_v, sem):
    wid = lax.axis_index("s") * NC + lax.axis_index("c")
    base = wid * PW
    pltpu.sync_copy(idx_hbm.at[pl.ds(base, PW)], idx_v)

    def chunk_body(c, carry):
        off = c * CK
        copies = []
        for j in range(NSG):
            isl = idx_v.at[pl.ds(off + j * SG, SG)]
            dsl = pl.ds(j * SG, SG)
            copies.append(pltpu.async_copy(table_hbm.at[isl], rowsb_v.at[dsl], sem))
            copies.append(pltpu.async_copy(u_hbm.at[isl], rowsu_v.at[dsl], sem))
        for cp in copies:
            cp.wait()
        pltpu.sync_copy(rowsb_v, outb_hbm.at[pl.ds(base + off, CK)])
        pltpu.sync_copy(rowsu_v, outu_hbm.at[pl.ds(base + off, CK)])
        return carry

    lax.fori_loop(0, NCH, chunk_body, 0)


_BLK = 2048


def _combine_body(u_ref, b_ref, w_ref, o_ref):
    o_ref[...] = b_ref[...] + jnp.dot(
        u_ref[...], w_ref[...], preferred_element_type=jnp.float32
    )


def kernel(x, base_table, pissa_U, pissa_S, pissa_V):
    idx = x.reshape(N)
    rows_b, rows_u = _sc_gather(base_table, pissa_U, idx)
    w = pissa_S[:, None] * pissa_V
    out = pl.pallas_call(
        _combine_body,
        grid=(N // _BLK,),
        in_specs=[
            pl.BlockSpec((_BLK, R), lambda i: (i, 0)),
            pl.BlockSpec((_BLK, DIM), lambda i: (i, 0)),
            pl.BlockSpec((R, DIM), lambda i: (0, 0)),
        ],
        out_specs=pl.BlockSpec((_BLK, DIM), lambda i: (i, 0)),
        out_shape=jax.ShapeDtypeStruct((N, DIM), jnp.float32),
    )(rows_u, rows_b, w)
    return out.reshape(B, L, DIM)


# speedup vs baseline: 1.2026x; 1.2026x over previous
import functools

import jax
import jax.numpy as jnp
from jax import lax
from jax.experimental import pallas as pl
from jax.experimental.pallas import tpu as pltpu
from jax.experimental.pallas import tpu_sc as plsc

VOCAB = 1000000
DIM = 64
R = 16
B = 16384
L = 20
N = B * L

_info = plsc.get_sparse_core_info()
NC = _info.num_cores
NS = _info.num_subcores
NW = NC * NS
PW = N // NW
CK = 1024
NCH = PW // CK
SG = 128
NSG = CK // SG

_mesh = plsc.VectorSubcoreMesh(core_axis_name="c", subcore_axis_name="s")


def _make_gather(width):
    @functools.partial(
        pl.kernel,
        mesh=_mesh,
        out_type=jax.ShapeDtypeStruct((N, width), jnp.float32),
        scratch_types=[
            pltpu.VMEM((PW,), jnp.int32),
            pltpu.VMEM((CK, width), jnp.float32),
            pltpu.SemaphoreType.DMA,
        ],
        compiler_params=pltpu.CompilerParams(use_tc_tiling_on_sc=False),
    )
    def _gather(table_hbm, idx_hbm, out_hbm, idx_v, rows_v, sem):
        wid = lax.axis_index("s") * NC + lax.axis_index("c")
        base = wid * PW
        pltpu.sync_copy(idx_hbm.at[pl.ds(base, PW)], idx_v)

        def chunk_body(c, carry):
            off = c * CK
            copies = []
            for j in range(NSG):
                isl = idx_v.at[pl.ds(off + j * SG, SG)]
                dsl = pl.ds(j * SG, SG)
                copies.append(pltpu.async_copy(table_hbm.at[isl], rows_v.at[dsl], sem))
            for cp in copies:
                cp.wait()
            pltpu.sync_copy(rows_v, out_hbm.at[pl.ds(base + off, CK)])
            return carry

        lax.fori_loop(0, NCH, chunk_body, 0)

    return _gather


_gather_base = _make_gather(DIM)
_gather_u = _make_gather(R)

_BLK = 2048
_NBB = B // _BLK


def _combine_body(u_ref, b_ref, w_ref, o_ref):
    rows = b_ref[...] + jnp.dot(
        u_ref[...], w_ref[...], preferred_element_type=jnp.float32
    )
    o_ref[...] = rows.T[None]


def kernel(x, base_table, pissa_U, pissa_S, pissa_V):
    idx = x.T.reshape(N)
    rows_b = _gather_base(base_table, idx)
    rows_u = _gather_u(pissa_U, idx)
    w = pissa_S[:, None] * pissa_V
    out_t = pl.pallas_call(
        _combine_body,
        grid=(L, _NBB),
        in_specs=[
            pl.BlockSpec((_BLK, R), lambda i, j: (i * _NBB + j, 0)),
            pl.BlockSpec((_BLK, DIM), lambda i, j: (i * _NBB + j, 0)),
            pl.BlockSpec((R, DIM), lambda i, j: (0, 0)),
        ],
        out_specs=pl.BlockSpec((1, DIM, _BLK), lambda i, j: (i, 0, j)),
        out_shape=jax.ShapeDtypeStruct((L, DIM, B), jnp.float32),
    )(rows_u, rows_b, w)
    return out_t.transpose(2, 0, 1)
